# baseline (speedup 1.0000x reference)
import functools

import jax
import jax.numpy as jnp
from jax import lax
from jax.experimental import pallas as pl

_N = 66
_K = 8
_H = 4
_DH = 12


def _fused_kernel(adj_ref, inten_ref, x_ref, t2v_ref, nb_ref,
                  w1_ref, b1_ref, w2_ref, b2_ref, wq_ref, wk_ref, wv_ref,
                  wo_ref, win_ref, wout_ref,
                  fin_ref, od_ref, indeg_ref, outdeg_ref):
    f32 = jnp.float32
    bdims = (((1,), (1,)), ((0,), (0,)))
    bdims2 = (((2,), (1,)), ((0,), (0,)))
    wmm = (((2,), (0,)), ((), ()))

    adj = adj_ref[...]
    e = jnp.exp(adj)
    ones8 = jnp.ones((_N, 8), f32)
    s = lax.dot_general(e, ones8, wmm, preferred_element_type=f32)[:, :, :1]
    a = (e >= 1e-8 * s).astype(f32)
    ns = lax.rsqrt(jnp.maximum(
        lax.dot_general(a, ones8, wmm, preferred_element_type=f32)[:, :, :1],
        1.0))
    nd = lax.rsqrt(jnp.maximum(
        lax.dot_general(a, ones8, (((1,), (0,)), ((), ())),
                        preferred_element_type=f32)[:, :, :1],
        1.0))

    x = x_ref[...]
    m1 = x * ns
    agg1 = lax.dot_general(a, m1, bdims, preferred_element_type=f32)
    h1 = lax.dot_general(agg1, w1_ref[...], wmm, preferred_element_type=f32)
    h = jnp.maximum(h1 * nd + b1_ref[...], 0.0)

    hw = lax.dot_general(h, w2_ref[...], wmm, preferred_element_type=f32)
    agg2 = lax.dot_general(a, hw * ns, bdims,
                           preferred_element_type=f32)
    node_emb = agg2 * nd + b2_ref[...]

    q = jnp.concatenate([node_emb, t2v_ref[...]], axis=-1)

    scale = f32(1.0) / jnp.sqrt(f32(_DH))
    qq = lax.dot_general(q, wq_ref[...] * scale, wmm,
                         preferred_element_type=f32)
    qk = lax.dot_general(q, wk_ref[...], wmm, preferred_element_type=f32)

    expi = jnp.exp(inten_ref[...])
    nb = nb_ref[...]
    col = lax.broadcasted_iota(jnp.int32, (_N, _N), 1)
    ei = jnp.zeros(adj.shape, f32)
    for k in range(_K):
        ohk = (nb[:, k:k + 1] == col).astype(f32)
        ei = ei + expi[:, :, k:k + 1] * ohk[None, :, :]

    hsel = (lax.broadcasted_iota(jnp.int32, (_H, _H * _DH), 1) // _DH ==
            lax.broadcasted_iota(jnp.int32, (_H, _H * _DH), 0)).astype(f32)
    wom = hsel * wo_ref[...]
    wvo = lax.dot_general(wom, wv_ref[...], (((1,), (1,)), ((), ())),
                          preferred_element_type=f32)
    vo = lax.dot_general(q, wvo, (((2,), (1,)), ((), ())),
                         preferred_element_type=f32)
    v5 = jnp.concatenate([vo, jnp.ones(vo.shape[:2] + (1,), f32)], axis=-1)

    fin_col = jnp.zeros(ns.shape, f32)
    for hh in range(_H):
        sl = slice(hh * _DH, (hh + 1) * _DH)
        fh = lax.dot_general(qq[:, :, sl], qk[:, :, sl],
                             (((2,), (2,)), ((0,), (0,))),
                             preferred_element_type=f32)
        w = jnp.exp(fh) * ei
        r = lax.dot_general(w, v5, bdims2, preferred_element_type=f32)
        fin_col = fin_col + r[:, :, hh:hh + 1] / r[:, :, _H:_H + 1]

    sig = f32(1.0) / (f32(1.0) + jnp.exp(-fin_col))
    fin_lane = jnp.swapaxes(fin_col, 1, 2)
    od_ref[...] = sig * fin_lane
    fin = fin_lane[:, 0, :]
    fin_ref[...] = fin
    indeg_ref[...] = jnp.dot(fin, win_ref[...], preferred_element_type=f32)
    outdeg_ref[...] = jnp.dot(fin, wout_ref[...], preferred_element_type=f32)


@functools.partial(jax.jit, static_argnames=("bb", "interpret"))
def _run(adj_matrix, intensity_score, input_data, time2vec_batch, nb,
         W1, b1r, W2, b2r, Wq, Wk, Wv, wo_r, weight_in, weight_out,
         bb=8, interpret=False):
    B = adj_matrix.shape[0]
    n = _N
    grid = (B // bb,)
    blk = lambda *shape: pl.BlockSpec(shape, lambda i: (i,) + (0,) * (len(shape) - 1))
    rep = lambda *shape: pl.BlockSpec(shape, lambda i: (0,) * len(shape))
    out_shape = [
        jax.ShapeDtypeStruct((B, n), jnp.float32),
        jax.ShapeDtypeStruct((B, n, n), jnp.float32),
        jax.ShapeDtypeStruct((B, n), jnp.float32),
        jax.ShapeDtypeStruct((B, n), jnp.float32),
    ]
    in_specs = [
        blk(bb, n, n),
        blk(bb, n, _K),
        blk(bb, n, input_data.shape[-1]),
        blk(bb, n, time2vec_batch.shape[-1]),
        rep(n, _K),
        rep(*W1.shape), rep(*b1r.shape), rep(*W2.shape), rep(*b2r.shape),
        rep(*Wq.shape), rep(*Wk.shape), rep(*Wv.shape), rep(*wo_r.shape),
        rep(*weight_in.shape), rep(*weight_out.shape),
    ]
    out_specs = [blk(bb, n), blk(bb, n, n), blk(bb, n), blk(bb, n)]
    return pl.pallas_call(
        _fused_kernel,
        grid=grid,
        in_specs=in_specs,
        out_specs=out_specs,
        out_shape=out_shape,
        interpret=interpret,
    )(adj_matrix, intensity_score, input_data, time2vec_batch, nb,
      W1, b1r, W2, b2r, Wq, Wk, Wv, wo_r, weight_in, weight_out)


def kernel(adj_matrix, intensity_score, input_data, time2vec_batch,
           neighbors_list, W1, b1, W2, b2, Wq, Wk, Wv, Wo,
           weight_in, weight_out):
    fin, od, indeg, outdeg = _run(
        adj_matrix, intensity_score, input_data, time2vec_batch,
        neighbors_list.astype(jnp.int32),
        W1, b1.reshape(1, -1), W2, b2.reshape(1, -1),
        Wq, Wk, Wv, Wo.reshape(1, -1), weight_in, weight_out,
        bb=32)
    return fin[:, :, None], od, indeg, outdeg


# speedup vs baseline: 1.0843x; 1.0843x over previous
import functools

import jax
import jax.numpy as jnp
from jax import lax
from jax.experimental import pallas as pl

_N = 66
_K = 8
_H = 4
_DH = 12


def _fused_kernel(adj_ref, inten_ref, x_ref, t2v_ref, nb_ref,
                  w1_ref, b1_ref, w2_ref, b2_ref, wq_ref, wk_ref, wv_ref,
                  wo_ref, win_ref, wout_ref,
                  fin_ref, od_ref, indeg_ref, outdeg_ref):
    f32 = jnp.float32
    bdims = (((1,), (1,)), ((0,), (0,)))
    bdims2 = (((2,), (1,)), ((0,), (0,)))
    wmm = (((2,), (0,)), ((), ()))

    adj = adj_ref[...]
    e = jnp.exp(adj)
    ones8 = jnp.ones((_N, 8), f32)
    s = lax.dot_general(e, ones8, wmm, preferred_element_type=f32)[:, :, :1]
    a = (e >= 1e-8 * s).astype(f32)
    ns = lax.rsqrt(jnp.maximum(
        lax.dot_general(a, ones8, wmm, preferred_element_type=f32)[:, :, :1],
        1.0))
    nd = lax.rsqrt(jnp.maximum(
        lax.dot_general(a, ones8, (((1,), (0,)), ((), ())),
                        preferred_element_type=f32)[:, :, :1],
        1.0))

    x = x_ref[...]
    m1 = x * ns
    agg1 = lax.dot_general(a, m1, bdims, preferred_element_type=f32)
    h1 = lax.dot_general(agg1, w1_ref[...], wmm, preferred_element_type=f32)
    h = jnp.maximum(h1 * nd + b1_ref[...], 0.0)

    hw = lax.dot_general(h, w2_ref[...], wmm, preferred_element_type=f32)
    agg2 = lax.dot_general(a, hw * ns, bdims,
                           preferred_element_type=f32)
    node_emb = agg2 * nd + b2_ref[...]

    q = jnp.concatenate([node_emb, t2v_ref[...]], axis=-1)

    scale = f32(1.0) / jnp.sqrt(f32(_DH))
    qq = lax.dot_general(q, wq_ref[...] * scale, wmm,
                         preferred_element_type=f32)
    qk = lax.dot_general(q, wk_ref[...], wmm, preferred_element_type=f32)

    expi = jnp.exp(inten_ref[...])
    nb = nb_ref[...]
    col = lax.broadcasted_iota(jnp.int32, (_N, _N), 1)
    ei = jnp.zeros(adj.shape, f32)
    for k in range(_K):
        ohk = (nb[:, k:k + 1] == col).astype(f32)
        ei = ei + expi[:, :, k:k + 1] * ohk[None, :, :]

    hsel = (lax.broadcasted_iota(jnp.int32, (_H, _H * _DH), 1) // _DH ==
            lax.broadcasted_iota(jnp.int32, (_H, _H * _DH), 0)).astype(f32)
    wom = hsel * wo_ref[...]
    wvo = lax.dot_general(wom, wv_ref[...], (((1,), (1,)), ((), ())),
                          preferred_element_type=f32)
    vo = lax.dot_general(wvo, q, (((1,), (2,)), ((), ())),
                         preferred_element_type=f32)

    fin_col = jnp.zeros(ns.shape, f32)
    for hh in range(_H):
        sl = slice(hh * _DH, (hh + 1) * _DH)
        fh = lax.dot_general(qq[:, :, sl], qk[:, :, sl],
                             (((2,), (2,)), ((0,), (0,))),
                             preferred_element_type=f32)
        w = jnp.exp(fh) * ei
        denom = jnp.sum(w, axis=-1, keepdims=True)
        numer = jnp.sum(w * vo[hh][:, None, :], axis=-1, keepdims=True)
        fin_col = fin_col + numer / denom

    sig = f32(1.0) / (f32(1.0) + jnp.exp(-fin_col))
    fin_lane = jnp.swapaxes(fin_col, 1, 2)
    od_ref[...] = sig * fin_lane
    fin = fin_lane[:, 0, :]
    fin_ref[...] = fin
    indeg_ref[...] = jnp.dot(fin, win_ref[...], preferred_element_type=f32)
    outdeg_ref[...] = jnp.dot(fin, wout_ref[...], preferred_element_type=f32)


@functools.partial(jax.jit, static_argnames=("bb", "interpret"))
def _run(adj_matrix, intensity_score, input_data, time2vec_batch, nb,
         W1, b1r, W2, b2r, Wq, Wk, Wv, wo_r, weight_in, weight_out,
         bb=8, interpret=False):
    B = adj_matrix.shape[0]
    n = _N
    grid = (B // bb,)
    blk = lambda *shape: pl.BlockSpec(shape, lambda i: (i,) + (0,) * (len(shape) - 1))
    rep = lambda *shape: pl.BlockSpec(shape, lambda i: (0,) * len(shape))
    out_shape = [
        jax.ShapeDtypeStruct((B, n), jnp.float32),
        jax.ShapeDtypeStruct((B, n, n), jnp.float32),
        jax.ShapeDtypeStruct((B, n), jnp.float32),
        jax.ShapeDtypeStruct((B, n), jnp.float32),
    ]
    in_specs = [
        blk(bb, n, n),
        blk(bb, n, _K),
        blk(bb, n, input_data.shape[-1]),
        blk(bb, n, time2vec_batch.shape[-1]),
        rep(n, _K),
        rep(*W1.shape), rep(*b1r.shape), rep(*W2.shape), rep(*b2r.shape),
        rep(*Wq.shape), rep(*Wk.shape), rep(*Wv.shape), rep(*wo_r.shape),
        rep(*weight_in.shape), rep(*weight_out.shape),
    ]
    out_specs = [blk(bb, n), blk(bb, n, n), blk(bb, n), blk(bb, n)]
    return pl.pallas_call(
        _fused_kernel,
        grid=grid,
        in_specs=in_specs,
        out_specs=out_specs,
        out_shape=out_shape,
        interpret=interpret,
    )(adj_matrix, intensity_score, input_data, time2vec_batch, nb,
      W1, b1r, W2, b2r, Wq, Wk, Wv, wo_r, weight_in, weight_out)


def kernel(adj_matrix, intensity_score, input_data, time2vec_batch,
           neighbors_list, W1, b1, W2, b2, Wq, Wk, Wv, Wo,
           weight_in, weight_out):
    fin, od, indeg, outdeg = _run(
        adj_matrix, intensity_score, input_data, time2vec_batch,
        neighbors_list.astype(jnp.int32),
        W1, b1.reshape(1, -1), W2, b2.reshape(1, -1),
        Wq, Wk, Wv, Wo.reshape(1, -1), weight_in, weight_out,
        bb=64)
    return fin[:, :, None], od, indeg, outdeg


# speedup vs baseline: 1.2757x; 1.1766x over previous
import functools

import jax
import jax.numpy as jnp
from jax import lax
from jax.experimental import pallas as pl

_N = 66
_K = 8
_H = 4
_DH = 12


def _fused_kernel(adj_ref, inten_ref, x_ref, t2v_ref, nb_ref,
                  w1_ref, b1_ref, w2_ref, b2_ref, wq_ref, wk_ref, wv_ref,
                  wo_ref, win_ref, wout_ref,
                  fin_ref, od_ref, indeg_ref, outdeg_ref):
    f32 = jnp.float32
    bdims = (((1,), (1,)), ((0,), (0,)))
    bdims2 = (((2,), (1,)), ((0,), (0,)))
    wmm = (((2,), (0,)), ((), ()))

    adj = adj_ref[...]
    e = jnp.exp(adj)
    s = jnp.sum(e, axis=-1, keepdims=True)
    a = (e >= 1e-8 * s).astype(f32)
    ns = jnp.expand_dims(
        lax.rsqrt(jnp.maximum(jnp.sum(a, axis=2), 1.0)), -1)
    nd = jnp.expand_dims(
        lax.rsqrt(jnp.maximum(jnp.sum(a, axis=1), 1.0)), -1)

    x = x_ref[...]
    m1 = x * ns
    agg1 = lax.dot_general(a, m1, bdims, preferred_element_type=f32)
    h1 = lax.dot_general(agg1, w1_ref[...], wmm, preferred_element_type=f32)
    h = jnp.maximum(h1 * nd + b1_ref[...], 0.0)

    hw = lax.dot_general(h, w2_ref[...], wmm, preferred_element_type=f32)
    agg2 = lax.dot_general(a, hw * ns, bdims,
                           preferred_element_type=f32)
    node_emb = agg2 * nd + b2_ref[...]

    q = jnp.concatenate([node_emb, t2v_ref[...]], axis=-1)

    scale = f32(1.0) / jnp.sqrt(f32(_DH))
    qq = lax.dot_general(q, wq_ref[...] * scale, wmm,
                         preferred_element_type=f32)
    qk = lax.dot_general(q, wk_ref[...], wmm, preferred_element_type=f32)

    expi = jnp.exp(inten_ref[...])
    nb = nb_ref[...]
    col = lax.broadcasted_iota(jnp.int32, (_N, _N), 1)
    ei = jnp.zeros(adj.shape, f32)
    for k in range(_K):
        ohk = (nb[:, k:k + 1] == col).astype(f32)
        ei = ei + expi[:, :, k:k + 1] * ohk[None, :, :]

    hsel = (lax.broadcasted_iota(jnp.int32, (_H, _H * _DH), 1) // _DH ==
            lax.broadcasted_iota(jnp.int32, (_H, _H * _DH), 0)).astype(f32)
    wom = hsel * wo_ref[...]
    wvo = lax.dot_general(wom, wv_ref[...], (((1,), (1,)), ((), ())),
                          preferred_element_type=f32)
    vo = lax.dot_general(wvo, q, (((1,), (2,)), ((), ())),
                         preferred_element_type=f32)

    fin_col = jnp.zeros(ns.shape, f32)
    for hh in range(_H):
        sl = slice(hh * _DH, (hh + 1) * _DH)
        fh = lax.dot_general(qq[:, :, sl], qk[:, :, sl],
                             (((2,), (2,)), ((0,), (0,))),
                             preferred_element_type=f32)
        w = jnp.exp(fh) * ei
        denom = jnp.sum(w, axis=-1, keepdims=True)
        numer = jnp.sum(w * vo[hh][:, None, :], axis=-1, keepdims=True)
        fin_col = fin_col + numer / denom

    sig = f32(1.0) / (f32(1.0) + jnp.exp(-fin_col))
    fin_lane = jnp.swapaxes(fin_col, 1, 2)
    od_ref[...] = sig * fin_lane
    fin = fin_lane[:, 0, :]
    fin_ref[...] = fin
    indeg_ref[...] = jnp.dot(fin, win_ref[...], preferred_element_type=f32)
    outdeg_ref[...] = jnp.dot(fin, wout_ref[...], preferred_element_type=f32)


@functools.partial(jax.jit, static_argnames=("bb", "interpret"))
def _run(adj_matrix, intensity_score, input_data, time2vec_batch, nb,
         W1, b1r, W2, b2r, Wq, Wk, Wv, wo_r, weight_in, weight_out,
         bb=8, interpret=False):
    B = adj_matrix.shape[0]
    n = _N
    grid = (B // bb,)
    blk = lambda *shape: pl.BlockSpec(shape, lambda i: (i,) + (0,) * (len(shape) - 1))
    rep = lambda *shape: pl.BlockSpec(shape, lambda i: (0,) * len(shape))
    out_shape = [
        jax.ShapeDtypeStruct((B, n), jnp.float32),
        jax.ShapeDtypeStruct((B, n, n), jnp.float32),
        jax.ShapeDtypeStruct((B, n), jnp.float32),
        jax.ShapeDtypeStruct((B, n), jnp.float32),
    ]
    in_specs = [
        blk(bb, n, n),
        blk(bb, n, _K),
        blk(bb, n, input_data.shape[-1]),
        blk(bb, n, time2vec_batch.shape[-1]),
        rep(n, _K),
        rep(*W1.shape), rep(*b1r.shape), rep(*W2.shape), rep(*b2r.shape),
        rep(*Wq.shape), rep(*Wk.shape), rep(*Wv.shape), rep(*wo_r.shape),
        rep(*weight_in.shape), rep(*weight_out.shape),
    ]
    out_specs = [blk(bb, n), blk(bb, n, n), blk(bb, n), blk(bb, n)]
    return pl.pallas_call(
        _fused_kernel,
        grid=grid,
        in_specs=in_specs,
        out_specs=out_specs,
        out_shape=out_shape,
        interpret=interpret,
    )(adj_matrix, intensity_score, input_data, time2vec_batch, nb,
      W1, b1r, W2, b2r, Wq, Wk, Wv, wo_r, weight_in, weight_out)


def kernel(adj_matrix, intensity_score, input_data, time2vec_batch,
           neighbors_list, W1, b1, W2, b2, Wq, Wk, Wv, Wo,
           weight_in, weight_out):
    fin, od, indeg, outdeg = _run(
        adj_matrix, intensity_score, input_data, time2vec_batch,
        neighbors_list.astype(jnp.int32),
        W1, b1.reshape(1, -1), W2, b2.reshape(1, -1),
        Wq, Wk, Wv, Wo.reshape(1, -1), weight_in, weight_out,
        bb=64)
    return fin[:, :, None], od, indeg, outdeg
